# baseline (speedup 1.0000x reference)
import functools

import jax
import jax.numpy as jnp
from jax import lax
from jax.experimental import pallas as pl
from jax.experimental.pallas import tpu as pltpu
from jax.experimental.pallas import tpu_sc as plsc

_B, _P, _D = 4096, 200, 64
_MAXLEN = 1000
_N = _B * _P
_NW = 32
_R = _N // _NW
_C = 256
_NCHUNK = _R // _C
_IDXW = 128
_KG = _C // _IDXW


def _sc_gather_add(pe2, idx2, x2):
    mesh = plsc.VectorSubcoreMesh(core_axis_name="c", subcore_axis_name="s")

    @functools.partial(
        pl.kernel,
        mesh=mesh,
        out_type=jax.ShapeDtypeStruct((_N // 2, 2 * _D), jnp.float32),
        scratch_types=[
            pltpu.VMEM((_KG, _IDXW), jnp.int32),
            pltpu.VMEM((_C, 128), jnp.float32),
            pltpu.VMEM((_C // 2, 2 * _D), jnp.float32),
            pltpu.SemaphoreType.DMA,
            pltpu.SemaphoreType.DMA,
        ],
    )
    def body(pe_hbm, idx_hbm, x_hbm, out_hbm, idx_v, rows_v, x_v, sem_g, sem_x):
        wid = lax.axis_index("s") * 2 + lax.axis_index("c")
        wbase = wid * _R

        def chunk_body(ci, carry):
            base = pl.multiple_of(wbase + ci * _C, _C)
            base2 = pl.multiple_of((wbase + ci * _C) // 2, _C // 2)
            for k in range(_KG):
                pltpu.sync_copy(
                    idx_hbm.at[pl.ds(base + k * _IDXW, _IDXW)], idx_v.at[k]
                )
            cx = pltpu.async_copy(x_hbm.at[pl.ds(base2, _C // 2)], x_v, sem_x)
            gathers = [
                pltpu.async_copy(
                    pe_hbm.at[idx_v.at[k]],
                    rows_v.at[pl.ds(k * _IDXW, _IDXW)],
                    sem_g,
                )
                for k in range(_KG)
            ]
            for g in gathers:
                g.wait()
            cx.wait()

            @plsc.parallel_loop(0, _C // 2, unroll=4)
            def row_add(p):
                for h in range(2):
                    for j in range(_D // 16):
                        plsc.addupdate(
                            x_v.at[p, pl.ds(h * _D + j * 16, 16)],
                            rows_v[2 * p + h, pl.ds(j * 16, 16)],
                        )

            pltpu.sync_copy(x_v, out_hbm.at[pl.ds(base2, _C // 2)])
            return carry

        lax.fori_loop(0, _NCHUNK, chunk_body, 0)

    return body(pe2, idx2, x2)


def kernel(x, indices, pe):
    x2 = x.reshape(_N // 2, 2 * _D)
    idx2 = indices.reshape(_N).astype(jnp.int32)
    pe2 = jnp.pad(pe.reshape(_MAXLEN, _D), ((0, 0), (0, 128 - _D)))
    out = _sc_gather_add(pe2, idx2, x2)
    return out.reshape(_B, _P, _D)


# speedup vs baseline: 1.3642x; 1.3642x over previous
import functools

import jax
import jax.numpy as jnp
from jax import lax
from jax.experimental import pallas as pl
from jax.experimental.pallas import tpu as pltpu
from jax.experimental.pallas import tpu_sc as plsc

_B, _P, _D = 4096, 200, 64
_MAXLEN = 1000
_N = _B * _P
_NW = 32
_BW = _B // _NW
_G1 = 128
_G2 = _P - _G1


def _sc_gather_add(pe2, idx2, x):
    mesh = plsc.VectorSubcoreMesh(core_axis_name="c", subcore_axis_name="s")

    @functools.partial(
        pl.kernel,
        mesh=mesh,
        compiler_params=pltpu.CompilerParams(use_tc_tiling_on_sc=True),
        out_type=jax.ShapeDtypeStruct((_B, _P, _D), jnp.float32),
        scratch_types=[
            pltpu.VMEM((_P,), jnp.int32),
            pltpu.VMEM((_P,), jnp.int32),
            pltpu.VMEM((_P, 128), jnp.float32),
            pltpu.VMEM((_P, 128), jnp.float32),
            pltpu.VMEM((_P, _D), jnp.float32),
            pltpu.VMEM((_P, _D), jnp.float32),
            pltpu.SemaphoreType.DMA,
            pltpu.SemaphoreType.DMA,
            pltpu.SemaphoreType.DMA,
            pltpu.SemaphoreType.DMA,
            pltpu.SemaphoreType.DMA,
            pltpu.SemaphoreType.DMA,
        ],
    )
    def body(pe_hbm, idx_hbm, x_hbm, out_hbm,
             idx0, idx1, rows0, rows1, xv0, xv1,
             sg0, sg1, sx0, sx1, ss0, ss1):
        wid = lax.axis_index("s") * 2 + lax.axis_index("c")
        bstart = wid * _BW
        bufs = ((idx0, rows0, xv0, sg0, sx0, ss0),
                (idx1, rows1, xv1, sg1, sx1, ss1))

        def do_chunk(b, idxv, rowsv, xv, sg, sx, ss):
            ib = pl.multiple_of(b * _P, 8)
            pltpu.sync_copy(idx_hbm.at[pl.ds(ib, _P)], idxv)
            g1 = pltpu.async_copy(
                pe_hbm.at[idxv.at[pl.ds(0, _G1)]], rowsv.at[pl.ds(0, _G1)], sg)
            g2 = pltpu.async_copy(
                pe_hbm.at[idxv.at[pl.ds(_G1, _G2)]], rowsv.at[pl.ds(_G1, _G2)], sg)
            cx = pltpu.async_copy(x_hbm.at[b], xv, sx)
            return g1, g2, cx

        def finish_chunk(b, idxv, rowsv, xv, sg, sx, ss, handles):
            g1, g2, cx = handles
            g1.wait()
            g2.wait()
            cx.wait()

            @plsc.parallel_loop(0, _P, unroll=4)
            def row_add(i):
                for j in range(_D // 16):
                    sl = pl.ds(j * 16, 16)
                    plsc.addupdate(xv.at[i, sl], rowsv[i, sl])

            return pltpu.async_copy(xv, out_hbm.at[b], ss)

        def pair_body(j, carry):
            @pl.when(j > 0)
            def _():
                for (idxv, rowsv, xv, sg, sx, ss) in bufs:
                    pltpu.make_async_copy(xv, out_hbm.at[bstart], ss).wait()

            b_even = bstart + 2 * j
            b_odd = b_even + 1
            h0 = do_chunk(b_even, *bufs[0])
            h1 = do_chunk(b_odd, *bufs[1])
            s0 = finish_chunk(b_even, *bufs[0], h0)
            s1 = finish_chunk(b_odd, *bufs[1], h1)
            del s0, s1
            return carry

        lax.fori_loop(0, _BW // 2, pair_body, 0)
        for (idxv, rowsv, xv, sg, sx, ss) in bufs:
            pltpu.make_async_copy(xv, out_hbm.at[bstart], ss).wait()

    return body(pe2, idx2, x)


def kernel(x, indices, pe):
    idx2 = indices.reshape(_N).astype(jnp.int32)
    pe2 = jnp.pad(pe.reshape(_MAXLEN, _D), ((0, 0), (0, 128 - _D)))
    return _sc_gather_add(pe2, idx2, x)


# speedup vs baseline: 1.7354x; 1.2720x over previous
import functools

import jax
import jax.numpy as jnp
from jax import lax
from jax.experimental import pallas as pl
from jax.experimental.pallas import tpu as pltpu
from jax.experimental.pallas import tpu_sc as plsc

_B, _P, _D = 4096, 200, 64
_MAXLEN = 1000
_NW = 32
_BS = 128
_NBT = _B // _BS
_NDT = _D // 8
_NPT = _P // 8
_NP = 2
_NL = 16
_NG = _BS // _NL


def _sc_gather_add(pe1, idx6, x5):
    mesh = plsc.VectorSubcoreMesh(core_axis_name="c", subcore_axis_name="s")

    @functools.partial(
        pl.kernel,
        mesh=mesh,
        compiler_params=pltpu.CompilerParams(needs_layout_passes=False),
        out_type=jax.ShapeDtypeStruct((_P, _NDT, _NBT, 8, _BS), jnp.float32),
        scratch_types=[
            pltpu.VMEM((_MAXLEN * _D,), jnp.float32),
            pltpu.VMEM((_NPT, 8, _BS), jnp.int32),
            pltpu.VMEM((_NP, _NDT, 8, _BS), jnp.float32),
            pltpu.VMEM((_NP, _NDT, 8, _BS), jnp.float32),
            pltpu.SemaphoreType.DMA,
            pltpu.SemaphoreType.DMA,
            pltpu.SemaphoreType.DMA,
            pltpu.SemaphoreType.DMA,
        ],
    )
    def body(pe_hbm, idx_hbm, x_hbm, out_hbm,
             pe_v, idx_v, xv0, xv1, sx0, sx1, ss0, ss1):
        wid = lax.axis_index("s") * 2 + lax.axis_index("c")
        pltpu.sync_copy(pe_hbm, pe_v)
        pltpu.sync_copy(idx_hbm.at[:, wid], idx_v)
        bufs = ((xv0, sx0, ss0), (xv1, sx1, ss1))

        def load_chunk(p0, xv, sx, ss):
            return pltpu.async_copy(
                x_hbm.at[pl.ds(p0, _NP), :, wid], xv, sx)

        def compute_chunk(p0, xv, sx, ss, handle):
            handle.wait()
            for pp in range(_NP):
                p = p0 + pp
                pt = p // 8
                ps = lax.rem(p, 8)
                for g in range(_NG):
                    iv = idx_v[pt, ps, pl.ds(g * _NL, _NL)] * _D

                    @plsc.parallel_loop(0, _D, unroll=8)
                    def dloop(d):
                        vals = plsc.load_gather(pe_v, [iv + d])
                        dt = d // 8
                        ds = lax.rem(d, 8)
                        plsc.addupdate(
                            xv.at[pp, dt, ds, pl.ds(g * _NL, _NL)], vals)

            return pltpu.async_copy(
                xv, out_hbm.at[pl.ds(p0, _NP), :, wid], ss)

        def pair_body(j, carry):
            @pl.when(j > 0)
            def _():
                for (xv, sx, ss) in bufs:
                    pltpu.make_async_copy(
                        xv, out_hbm.at[pl.ds(0, _NP), :, wid], ss).wait()

            p_even = 2 * _NP * j
            h0 = load_chunk(p_even, *bufs[0])
            h1 = load_chunk(p_even + _NP, *bufs[1])
            compute_chunk(p_even, *bufs[0], h0)
            compute_chunk(p_even + _NP, *bufs[1], h1)
            return carry

        lax.fori_loop(0, _P // (2 * _NP), pair_body, 0)
        for (xv, sx, ss) in bufs:
            pltpu.make_async_copy(
                xv, out_hbm.at[pl.ds(0, _NP), :, wid], ss).wait()

    return body(pe1, idx6, x5)


def kernel(x, indices, pe):
    x5 = jnp.transpose(
        x.reshape(_NBT, _BS, _P, _NDT, 8), (2, 3, 0, 4, 1))
    idx6 = jnp.transpose(
        indices.reshape(_NBT, _BS, _NPT, 8), (2, 0, 3, 1)).astype(jnp.int32)
    pe1 = pe.reshape(_MAXLEN * _D)
    out5 = _sc_gather_add(pe1, idx6, x5)
    return jnp.transpose(out5, (2, 4, 0, 1, 3)).reshape(_B, _P, _D)
